# baseline (speedup 1.0000x reference)
import functools

import jax
import jax.numpy as jnp
from jax import lax
from jax.experimental import pallas as pl
from jax.experimental.pallas import tpu as pltpu
from jax.experimental.pallas import tpu_sc as plsc

NUM_SEGMENTS = 128
D_FEAT = 128
N_ROWS = 100000
WIN = 80
NUM_WINDOWS = N_ROWS // WIN
NC = 2
NS = 16
NW = NC * NS
SLOTS = (NUM_WINDOWS + NW - 1) // NW
ROWS_PER_TILE = NUM_SEGMENTS // NS


def _sc_segment_sums(feat, ids2d):
    mesh = plsc.VectorSubcoreMesh(core_axis_name="c", subcore_axis_name="s")

    @functools.partial(
        pl.kernel,
        out_type=(
            jax.ShapeDtypeStruct((NC, NUM_SEGMENTS, D_FEAT), jnp.float32),
            jax.ShapeDtypeStruct((NW, NUM_SEGMENTS), jnp.float32),
        ),
        mesh=mesh,
        compiler_params=pltpu.CompilerParams(
            use_tc_tiling_on_sc=False, needs_layout_passes=False),
        scratch_types=[
            pltpu.VMEM((WIN, D_FEAT), jnp.float32),
            pltpu.VMEM((WIN, D_FEAT), jnp.float32),
            pltpu.VMEM((SLOTS, WIN), jnp.int32),
            pltpu.VMEM((NUM_SEGMENTS,), jnp.float32),
            pltpu.VMEM_SHARED((NUM_SEGMENTS, D_FEAT), jnp.float32),
            pltpu.SemaphoreType.DMA,
            pltpu.SemaphoreType.DMA,
            pltpu.SemaphoreType.DMA,
            pltpu.SemaphoreType.DMA,
        ],
    )
    def seg_sum(feat_hbm, ids_hbm, out_sum, out_cnt,
                fbuf_a, fbuf_b, idx_all, cnt_buf, acc_sh,
                sem_a, sem_b, sem_sa, sem_sb):
        c = lax.axis_index("c")
        s = lax.axis_index("s")
        w = s * NC + c

        n_w = jnp.where(w < 2, SLOTS, SLOTS - 1)
        b_w = (SLOTS - 1) * w + jnp.minimum(w, 2)
        pb = jnp.minimum(b_w, NUM_WINDOWS - SLOTS)
        shift = b_w - pb

        pltpu.sync_copy(ids_hbm.at[pl.ds(pb, SLOTS)], idx_all)

        z16 = jnp.zeros((16,), jnp.float32)
        for j in range(NUM_SEGMENTS // 16):
            cnt_buf[pl.ds(j * 16, 16)] = z16
        for i in range(ROWS_PER_TILE):
            for j in range(D_FEAT // 16):
                fbuf_a[i, pl.ds(j * 16, 16)] = z16
        pltpu.sync_copy(fbuf_a.at[pl.ds(0, ROWS_PER_TILE)],
                        acc_sh.at[pl.ds(s * ROWS_PER_TILE, ROWS_PER_TILE)])
        plsc.subcore_barrier()

        def win_base(l):
            return jnp.minimum(b_w + l, NUM_WINDOWS - 1) * WIN

        def gather(l, buf, sem):
            pltpu.make_async_copy(
                feat_hbm.at[pl.ds(win_base(l), WIN)], buf, sem).start()

        def wait(l, buf, sem):
            pltpu.make_async_copy(
                feat_hbm.at[pl.ds(win_base(l), WIN)], buf, sem).wait()

        ones16 = jnp.ones((16,), jnp.float32)

        def scatter_desc(l, buf, sem):
            return pltpu.make_async_copy(buf, acc_sh.at[idx_all.at[shift + l]],
                                         sem)

        def process(l, buf, gsem, ssem):
            wait(l, buf, gsem)

            @pl.when(l < n_w)
            def _():
                scatter_desc(l, buf, ssem).start(add=True)
                for k in range(WIN // 16):
                    ids16 = idx_all[shift + l, pl.ds(k * 16, 16)]
                    plsc.addupdate_scatter(cnt_buf, [ids16], ones16)

        def drain_scatter(l, buf, ssem):
            scatter_desc(l, buf, ssem).wait()

        gather(0, fbuf_a, sem_a)
        gather(1, fbuf_b, sem_b)

        def body(i, carry):
            l0 = 2 * i
            l1 = 2 * i + 1

            process(l0, fbuf_a, sem_a, sem_sa)
            process(l1, fbuf_b, sem_b, sem_sb)

            @pl.when(l0 + 2 < SLOTS)
            def _():
                drain_scatter(l0, fbuf_a, sem_sa)
                gather(l0 + 2, fbuf_a, sem_a)

            @pl.when(l1 + 2 < SLOTS)
            def _():
                drain_scatter(l1, fbuf_b, sem_sb)
                gather(l1 + 2, fbuf_b, sem_b)

            return carry

        lax.fori_loop(0, SLOTS // 2, body, 0)

        drain_scatter(SLOTS - 2, fbuf_a, sem_sa)

        @pl.when(n_w == SLOTS)
        def _():
            drain_scatter(SLOTS - 1, fbuf_b, sem_sb)

        pltpu.sync_copy(cnt_buf, out_cnt.at[w])
        plsc.subcore_barrier()

        @pl.when(s == 0)
        def _():
            pltpu.sync_copy(acc_sh, out_sum.at[c])

    return seg_sum(feat, ids2d)


def _merge_and_divide(sums, cnts):
    def combine(sum_ref, cnt_ref, out_ref):
        total = sum_ref[0] + sum_ref[1]
        cnt = jnp.sum(cnt_ref[...], axis=0)
        denom = jnp.maximum(cnt, 1.0)[:, None]
        out_ref[...] = total / denom

    return pl.pallas_call(
        combine,
        out_shape=jax.ShapeDtypeStruct((NUM_SEGMENTS, D_FEAT), jnp.float32),
    )(sums, cnts)


@jax.jit
def kernel(feat, segment_ids):
    ids2d = segment_ids.astype(jnp.int32).reshape(NUM_WINDOWS, WIN)
    sums, cnts = _sc_segment_sums(feat, ids2d)
    return _merge_and_divide(sums, cnts)
